# baseline (speedup 1.0000x reference)
import functools

import jax
import jax.numpy as jnp
from jax import lax
from jax.experimental import pallas as pl
from jax.experimental.pallas import tpu as pltpu
from jax.experimental.pallas import tpu_sc as plsc

D_MODEL = 64
N = 1000000
K = 16
BLK = 8192
G = -(-N // BLK)

_NEG_INF = float("-inf")
_BIG_I32 = 2**31 - 1


def _sims_topk_body(q_ref, k_ref, sims_out, idx_out, s_scr, bmax_s, bidx_s):
    i = pl.program_id(0)

    q = q_ref[...]
    qn = q / jnp.maximum(jnp.sqrt(jnp.sum(q * q)), 1e-12)

    kblk = k_ref[...]
    dots = jnp.sum(kblk * qn, axis=0, keepdims=True)
    nrm2 = jnp.sum(kblk * kblk, axis=0, keepdims=True)
    s3 = dots / jnp.maximum(jnp.sqrt(nrm2), 1e-12)
    lin = lax.broadcasted_iota(jnp.int32, (1, BLK), 1)
    s3 = jnp.where(i * BLK + lin < N, s3, _NEG_INF)
    s_scr[pl.ds(i, 1)] = s3

    m = jnp.max(s3)
    lidx = jnp.min(jnp.where(s3 == m, lin, _BIG_I32))
    bmax_s[i] = m
    bidx_s[i] = i * BLK + lidx

    @pl.when(i == G - 1)
    def _final():
        for j in range(K):
            def scan_blocks(g, carry):
                bv, bg = carry
                v = bmax_s[g]
                take = v > bv
                return (jnp.where(take, v, bv),
                        jnp.where(take, g, bg))
            bestv, bestg = lax.fori_loop(
                0, G, scan_blocks, (jnp.float32(_NEG_INF), jnp.int32(0)))
            besti = bidx_s[bestg]
            sims_out[j] = bestv
            idx_out[j] = besti

            row = s_scr[pl.ds(bestg, 1)]
            off = besti - bestg * BLK
            row = jnp.where(lin == off, _NEG_INF, row)
            s_scr[pl.ds(bestg, 1)] = row
            nm = jnp.max(row)
            nidx = jnp.min(jnp.where(row == nm, lin, _BIG_I32))
            bmax_s[bestg] = nm
            bidx_s[bestg] = bestg * BLK + nidx


def _sims_topk(query, keys_t):
    return pl.pallas_call(
        _sims_topk_body,
        grid=(G,),
        in_specs=[
            pl.BlockSpec((D_MODEL, 1), lambda i: (0, 0)),
            pl.BlockSpec((D_MODEL, BLK), lambda i: (0, i)),
        ],
        out_specs=[
            pl.BlockSpec(memory_space=pltpu.SMEM),
            pl.BlockSpec(memory_space=pltpu.SMEM),
        ],
        out_shape=[
            jax.ShapeDtypeStruct((K,), jnp.float32),
            jax.ShapeDtypeStruct((K,), jnp.int32),
        ],
        scratch_shapes=[
            pltpu.VMEM((G, BLK), jnp.float32),
            pltpu.SMEM((G,), jnp.float32),
            pltpu.SMEM((G,), jnp.int32),
        ],
    )(query.reshape(D_MODEL, 1), keys_t)


def _sc_gather(vflat, idx):
    mesh = plsc.VectorSubcoreMesh(core_axis_name="c", subcore_axis_name="s")

    @functools.partial(
        pl.kernel,
        mesh=mesh,
        out_type=jax.ShapeDtypeStruct((D_MODEL, K), jnp.float32),
        scratch_types=[
            pltpu.VMEM((K,), jnp.int32),
            pltpu.VMEM((D_MODEL, K), jnp.int32),
            pltpu.VMEM((D_MODEL, K), jnp.float32),
            pltpu.SemaphoreType.DMA,
        ],
    )
    def gather_k(vflat_hbm, idx_hbm, out_hbm, idx_v, idxbuf_v, rows_v, sem):
        c = lax.axis_index("c")
        s = lax.axis_index("s")
        wid = s * 2 + c

        @pl.when(wid == 0)
        def _():
            pltpu.sync_copy(idx_hbm, idx_v)
            base = idx_v[...]
            for d in range(D_MODEL):
                idxbuf_v[d] = base + d * N
            handles = []
            for d in range(D_MODEL):
                handles.append(pltpu.async_copy(
                    vflat_hbm.at[idxbuf_v.at[d]], rows_v.at[d], sem))
            for h in handles:
                h.wait()
            pltpu.sync_copy(rows_v, out_hbm)

    return gather_k(vflat, idx)


def kernel(query, keys, values, k):
    topk_sims, topk_idx = _sims_topk(query, keys.T)
    valid = jnp.arange(K) < k
    topk_sims = jnp.where(valid, topk_sims, topk_sims[0])
    topk_idx = jnp.where(valid, topk_idx, topk_idx[0])
    vals_t = _sc_gather(values.T.reshape(-1), topk_idx)
    return vals_t.T, topk_sims


# speedup vs baseline: 23.7178x; 23.7178x over previous
import functools

import jax
import jax.numpy as jnp
from jax import lax
from jax.experimental import pallas as pl
from jax.experimental.pallas import tpu as pltpu
from jax.experimental.pallas import tpu_sc as plsc

D_MODEL = 64
N = 1000000
K = 16
BLK = 8192
G = -(-N // BLK)

_NEG_INF = float("-inf")
_BIG_I32 = 2**31 - 1


def _sims_topk_body(q_ref, k_ref, sims_out, idx_out, s_scr, bmax_s, bidx_s):
    i = pl.program_id(0)

    q = q_ref[...]
    qn = q / jnp.maximum(jnp.sqrt(jnp.sum(q * q)), 1e-12)

    kblk = k_ref[...]
    dots = jnp.sum(kblk * qn, axis=0, keepdims=True)
    nrm2 = jnp.sum(kblk * kblk, axis=0, keepdims=True)
    s3 = dots / jnp.maximum(jnp.sqrt(nrm2), 1e-12)
    lin = lax.broadcasted_iota(jnp.int32, (1, BLK), 1)
    s3 = jnp.where(i * BLK + lin < N, s3, _NEG_INF)
    s_scr[pl.ds(i, 1)] = s3

    m = jnp.max(s3)
    lidx = jnp.min(jnp.where(s3 == m, lin, _BIG_I32))
    bmax_s[i] = m
    bidx_s[i] = i * BLK + lidx

    @pl.when(i == G - 1)
    def _final():
        for j in range(K):
            def scan_blocks(g, carry):
                bv, bg = carry
                v = bmax_s[g]
                take = v > bv
                return (jnp.where(take, v, bv),
                        jnp.where(take, g, bg))
            bestv, bestg = lax.fori_loop(
                0, G, scan_blocks, (jnp.float32(_NEG_INF), jnp.int32(0)))
            besti = bidx_s[bestg]
            sims_out[j] = bestv
            idx_out[j] = besti

            row = s_scr[pl.ds(bestg, 1)]
            off = besti - bestg * BLK
            row = jnp.where(lin == off, _NEG_INF, row)
            s_scr[pl.ds(bestg, 1)] = row
            nm = jnp.max(row)
            nidx = jnp.min(jnp.where(row == nm, lin, _BIG_I32))
            bmax_s[bestg] = nm
            bidx_s[bestg] = bestg * BLK + nidx


def _sims_topk(query, keys_t):
    return pl.pallas_call(
        _sims_topk_body,
        grid=(G,),
        in_specs=[
            pl.BlockSpec((D_MODEL, 1), lambda i: (0, 0)),
            pl.BlockSpec((D_MODEL, BLK), lambda i: (0, i)),
        ],
        out_specs=[
            pl.BlockSpec(memory_space=pltpu.SMEM),
            pl.BlockSpec(memory_space=pltpu.SMEM),
        ],
        out_shape=[
            jax.ShapeDtypeStruct((K,), jnp.float32),
            jax.ShapeDtypeStruct((K,), jnp.int32),
        ],
        scratch_shapes=[
            pltpu.VMEM((G, BLK), jnp.float32),
            pltpu.SMEM((G,), jnp.float32),
            pltpu.SMEM((G,), jnp.int32),
        ],
    )(query.reshape(D_MODEL, 1), keys_t)


def _gather_body(idx_sref, vblk_ref, out_ref):
    j = pl.program_id(0)
    r = idx_sref[j]
    lane = lax.broadcasted_iota(jnp.int32, (D_MODEL, 128), 1)
    col = jnp.sum(jnp.where(lane == r % 128, vblk_ref[...], 0.0),
                  axis=1, keepdims=True)
    outlane = lax.broadcasted_iota(jnp.int32, (D_MODEL, K), 1)
    out_ref[...] = jnp.where(outlane == j, col, out_ref[...])


def _gather_cols(values_t, idx):
    grid_spec = pltpu.PrefetchScalarGridSpec(
        num_scalar_prefetch=1,
        grid=(K,),
        in_specs=[
            pl.BlockSpec((D_MODEL, 128), lambda j, idx_ref: (0, idx_ref[j] // 128)),
        ],
        out_specs=pl.BlockSpec((D_MODEL, K), lambda j, idx_ref: (0, 0)),
    )
    return pl.pallas_call(
        _gather_body,
        grid_spec=grid_spec,
        out_shape=jax.ShapeDtypeStruct((D_MODEL, K), jnp.float32),
    )(idx, values_t)


def kernel(query, keys, values, k):
    topk_sims, topk_idx = _sims_topk(query, keys.T)
    valid = jnp.arange(K) < k
    topk_sims = jnp.where(valid, topk_sims, topk_sims[0])
    topk_idx = jnp.where(valid, topk_idx, topk_idx[0])
    vals_t = _gather_cols(values.T, topk_idx)
    return vals_t.T, topk_sims


# speedup vs baseline: 23.7589x; 1.0017x over previous
import functools

import jax
import jax.numpy as jnp
from jax import lax
from jax.experimental import pallas as pl
from jax.experimental.pallas import tpu as pltpu
from jax.experimental.pallas import tpu_sc as plsc

D_MODEL = 64
N = 1000000
K = 16
BLK = 8192
G = -(-N // BLK)
TOT = G * BLK
NW = 32
CH = TOT // NW
GRP = 48
NGRP = CH // (GRP * 16)
NGPAD = 48

_NEG_INF = float("-inf")
_BIG_I32 = 2**31 - 1


def _sims_body(q_ref, k_ref, s_out):
    i = pl.program_id(0)
    q = q_ref[...]
    qn = q / jnp.maximum(jnp.sqrt(jnp.sum(q * q)), 1e-12)
    kblk = k_ref[...]
    dots = jnp.sum(kblk * qn, axis=0, keepdims=True)
    nrm2 = jnp.sum(kblk * kblk, axis=0, keepdims=True)
    s3 = dots / jnp.maximum(jnp.sqrt(nrm2), 1e-12)
    lin = lax.broadcasted_iota(jnp.int32, (1, BLK), 1)
    s3 = jnp.where(i * BLK + lin < N, s3, _NEG_INF)
    s_out[...] = s3.reshape(BLK)


def _sims(query, keys_t):
    return pl.pallas_call(
        _sims_body,
        grid=(G,),
        in_specs=[
            pl.BlockSpec((D_MODEL, 1), lambda i: (0, 0)),
            pl.BlockSpec((D_MODEL, BLK), lambda i: (0, i)),
        ],
        out_specs=pl.BlockSpec((BLK,), lambda i: (i,)),
        out_shape=jax.ShapeDtypeStruct((TOT,), jnp.float32),
    )(query.reshape(D_MODEL, 1), keys_t)


def _sc_topk_local(sims):
    mesh = plsc.VectorSubcoreMesh(core_axis_name="c", subcore_axis_name="s")

    @functools.partial(
        pl.kernel,
        mesh=mesh,
        out_type=[
            jax.ShapeDtypeStruct((NW * K,), jnp.float32),
            jax.ShapeDtypeStruct((NW * K,), jnp.int32),
        ],
        scratch_types=[
            pltpu.VMEM((CH,), jnp.float32),
            pltpu.VMEM((NGPAD,), jnp.float32),
            pltpu.VMEM((K,), jnp.float32),
            pltpu.VMEM((K,), jnp.int32),
        ],
        compiler_params=pltpu.CompilerParams(needs_layout_passes=False),
    )
    def topk_local(sims_hbm, cv_hbm, ci_hbm, buf_v, gmax_v, ov_v, oi_v):
        c = lax.axis_index("c")
        s = lax.axis_index("s")
        wid = s * 2 + c
        base = wid * CH
        pltpu.sync_copy(sims_hbm.at[pl.ds(base, CH)], buf_v)

        iota = lax.iota(jnp.int32, 16)
        ninf = jnp.full((16,), _NEG_INF, jnp.float32)

        for v in range(NGPAD // 16):
            gmax_v[pl.ds(v * 16, 16)] = ninf

        def group_max(g):
            def inner(cc, macc):
                return jnp.maximum(macc, buf_v[pl.ds(g * GRP * 16 + cc * 16, 16)])
            return jnp.max(lax.fori_loop(0, GRP, inner, ninf))

        def phase1(g, carry):
            gm = group_max(g)
            gb = (g // 16) * 16
            cur = gmax_v[pl.ds(gb, 16)]
            gmax_v[pl.ds(gb, 16)] = jnp.where(iota == g - gb, gm, cur)
            return carry

        lax.fori_loop(0, NGRP, phase1, jnp.int32(0))

        outv = ninf
        outi = jnp.zeros((16,), jnp.int32)
        for t in range(K):
            g0 = gmax_v[pl.ds(0, 16)]
            g1 = gmax_v[pl.ds(16, 16)]
            g2 = gmax_v[pl.ds(32, 16)]
            gm = jnp.max(jnp.maximum(g0, jnp.maximum(g1, g2)))
            cand0 = jnp.where(g0 == gm, iota, _BIG_I32)
            cand1 = jnp.where(g1 == gm, iota + 16, _BIG_I32)
            cand2 = jnp.where(g2 == gm, iota + 32, _BIG_I32)
            gstar = jnp.min(jnp.minimum(cand0, jnp.minimum(cand1, cand2)))

            def rescan(cc, carry):
                m, mc = carry
                v = buf_v[pl.ds(gstar * GRP * 16 + cc * 16, 16)]
                upd = v > m
                return (jnp.where(upd, v, m), jnp.where(upd, cc, mc))

            m, mc = lax.fori_loop(0, GRP, rescan,
                                  (ninf, jnp.zeros((16,), jnp.int32)))
            gm2 = jnp.max(m)
            pe = jnp.min(jnp.where(
                m == gm2, (gstar * GRP + mc) * 16 + iota, _BIG_I32))
            outv = jnp.where(iota == t, gm2, outv)
            outi = jnp.where(iota == t, base + pe, outi)

            pb = (pe // 16) * 16
            cur = buf_v[pl.ds(pb, 16)]
            buf_v[pl.ds(pb, 16)] = jnp.where(iota == pe - pb, _NEG_INF, cur)

            ngm = group_max(gstar)
            gb = (gstar // 16) * 16
            cur = gmax_v[pl.ds(gb, 16)]
            gmax_v[pl.ds(gb, 16)] = jnp.where(iota == gstar - gb, ngm, cur)

        ov_v[...] = outv
        oi_v[...] = outi
        pltpu.sync_copy(ov_v, cv_hbm.at[pl.ds(wid * K, K)])
        pltpu.sync_copy(oi_v, ci_hbm.at[pl.ds(wid * K, K)])

    return topk_local(sims)


def _sc_topk_merge(cand_v, cand_i):
    mesh = plsc.VectorSubcoreMesh(core_axis_name="c", subcore_axis_name="s")
    NC = NW * K // 16

    @functools.partial(
        pl.kernel,
        mesh=mesh,
        out_type=[
            jax.ShapeDtypeStruct((K,), jnp.float32),
            jax.ShapeDtypeStruct((K,), jnp.int32),
        ],
        scratch_types=[
            pltpu.VMEM((NW * K,), jnp.float32),
            pltpu.VMEM((NW * K,), jnp.int32),
            pltpu.VMEM((K,), jnp.float32),
            pltpu.VMEM((K,), jnp.int32),
        ],
        compiler_params=pltpu.CompilerParams(needs_layout_passes=False),
    )
    def topk_merge(cv_hbm, ci_hbm, tv_hbm, ti_hbm, cv_v, ci_v, ov_v, oi_v):
        c = lax.axis_index("c")
        s = lax.axis_index("s")
        wid = s * 2 + c

        @pl.when(wid == 0)
        def _():
            pltpu.sync_copy(cv_hbm, cv_v)
            pltpu.sync_copy(ci_hbm, ci_v)
            iota = lax.iota(jnp.int32, 16)
            ninf = jnp.full((16,), _NEG_INF, jnp.float32)
            outv = ninf
            outi = jnp.zeros((16,), jnp.int32)
            for t in range(K):
                def vmax(cc, macc):
                    return jnp.maximum(macc, cv_v[pl.ds(cc * 16, 16)])
                gm = jnp.max(lax.fori_loop(0, NC, vmax, ninf))

                def imin(cc, micc):
                    vv = cv_v[pl.ds(cc * 16, 16)]
                    iv = ci_v[pl.ds(cc * 16, 16)]
                    return jnp.minimum(micc, jnp.where(vv == gm, iv, _BIG_I32))
                gi = jnp.min(lax.fori_loop(
                    0, NC, imin, jnp.full((16,), _BIG_I32, jnp.int32)))

                def mask(cc, carry):
                    vv = cv_v[pl.ds(cc * 16, 16)]
                    iv = ci_v[pl.ds(cc * 16, 16)]
                    hit = jnp.logical_and(vv == gm, iv == gi)
                    cv_v[pl.ds(cc * 16, 16)] = jnp.where(hit, _NEG_INF, vv)
                    return carry
                lax.fori_loop(0, NC, mask, jnp.int32(0))

                outv = jnp.where(iota == t, gm, outv)
                outi = jnp.where(iota == t, gi, outi)
            ov_v[...] = outv
            oi_v[...] = outi
            pltpu.sync_copy(ov_v, tv_hbm)
            pltpu.sync_copy(oi_v, ti_hbm)

    return topk_merge(cand_v, cand_i)


def _gather_body(idx_sref, vblk_ref, out_ref):
    j = pl.program_id(0)
    r = idx_sref[j]
    lane = lax.broadcasted_iota(jnp.int32, (D_MODEL, 128), 1)
    col = jnp.sum(jnp.where(lane == r % 128, vblk_ref[...], 0.0),
                  axis=1, keepdims=True)
    outlane = lax.broadcasted_iota(jnp.int32, (D_MODEL, K), 1)
    out_ref[...] = jnp.where(outlane == j, col, out_ref[...])


def _gather_cols(values_t, idx):
    grid_spec = pltpu.PrefetchScalarGridSpec(
        num_scalar_prefetch=1,
        grid=(K,),
        in_specs=[
            pl.BlockSpec((D_MODEL, 128), lambda j, idx_ref: (0, idx_ref[j] // 128)),
        ],
        out_specs=pl.BlockSpec((D_MODEL, K), lambda j, idx_ref: (0, 0)),
    )
    return pl.pallas_call(
        _gather_body,
        grid_spec=grid_spec,
        out_shape=jax.ShapeDtypeStruct((D_MODEL, K), jnp.float32),
    )(idx, values_t)


def kernel(query, keys, values, k):
    sims = _sims(query, keys.T)
    cand_v, cand_i = _sc_topk_local(sims)
    topk_sims, topk_idx = _sc_topk_merge(cand_v, cand_i)
    valid = jnp.arange(K) < k
    topk_sims = jnp.where(valid, topk_sims, topk_sims[0])
    topk_idx = jnp.where(valid, topk_idx, topk_idx[0])
    vals_t = _gather_cols(values.T, topk_idx)
    return vals_t.T, topk_sims


# speedup vs baseline: 27.6759x; 1.1649x over previous
import functools

import jax
import jax.numpy as jnp
from jax import lax
from jax.experimental import pallas as pl
from jax.experimental.pallas import tpu as pltpu
from jax.experimental.pallas import tpu_sc as plsc

D_MODEL = 64
N = 1000000
K = 16
BLK = 16384
G = -(-N // BLK)
TOT = G * BLK
NW = 32
CH = TOT // NW
GRP = 62
NGRP = CH // (GRP * 16)
NGPAD = NGRP

_NEG_INF = float("-inf")
_BIG_I32 = 2**31 - 1


def _sims_body(q_ref, k_ref, s_out):
    i = pl.program_id(0)
    q = q_ref[...]
    qn = q / jnp.maximum(jnp.sqrt(jnp.sum(q * q)), 1e-12)
    kblk = k_ref[...]
    dots = jnp.sum(kblk * qn, axis=0, keepdims=True)
    nrm2 = jnp.sum(kblk * kblk, axis=0, keepdims=True)
    s3 = dots / jnp.maximum(jnp.sqrt(nrm2), 1e-12)
    lin = lax.broadcasted_iota(jnp.int32, (1, BLK), 1)
    s3 = jnp.where(i * BLK + lin < N, s3, _NEG_INF)
    s_out[...] = s3.reshape(BLK)


def _sims(query, keys_t):
    return pl.pallas_call(
        _sims_body,
        grid=(G,),
        in_specs=[
            pl.BlockSpec((D_MODEL, 1), lambda i: (0, 0)),
            pl.BlockSpec((D_MODEL, BLK), lambda i: (0, i)),
        ],
        out_specs=pl.BlockSpec((BLK,), lambda i: (i,)),
        out_shape=jax.ShapeDtypeStruct((TOT,), jnp.float32),
    )(query.reshape(D_MODEL, 1), keys_t)


def _sc_topk_local(sims):
    mesh = plsc.VectorSubcoreMesh(core_axis_name="c", subcore_axis_name="s")

    @functools.partial(
        pl.kernel,
        mesh=mesh,
        out_type=[
            jax.ShapeDtypeStruct((NW * K,), jnp.float32),
            jax.ShapeDtypeStruct((NW * K,), jnp.int32),
        ],
        scratch_types=[
            pltpu.VMEM((CH,), jnp.float32),
            pltpu.VMEM((NGPAD,), jnp.float32),
            pltpu.VMEM((K,), jnp.float32),
            pltpu.VMEM((K,), jnp.int32),
        ],
        compiler_params=pltpu.CompilerParams(needs_layout_passes=False),
    )
    def topk_local(sims_hbm, cv_hbm, ci_hbm, buf_v, gmax_v, ov_v, oi_v):
        c = lax.axis_index("c")
        s = lax.axis_index("s")
        wid = s * 2 + c
        base = wid * CH
        pltpu.sync_copy(sims_hbm.at[pl.ds(base, CH)], buf_v)

        iota = lax.iota(jnp.int32, 16)
        ninf = jnp.full((16,), _NEG_INF, jnp.float32)

        for v in range(NGPAD // 16):
            gmax_v[pl.ds(v * 16, 16)] = ninf

        def group_max(g):
            def inner(cc, macc):
                return jnp.maximum(macc, buf_v[pl.ds(g * GRP * 16 + cc * 16, 16)])
            return jnp.max(lax.fori_loop(0, GRP, inner, ninf))

        def phase1(g, carry):
            gm = group_max(g)
            gb = (g // 16) * 16
            cur = gmax_v[pl.ds(gb, 16)]
            gmax_v[pl.ds(gb, 16)] = jnp.where(iota == g - gb, gm, cur)
            return carry

        lax.fori_loop(0, NGRP, phase1, jnp.int32(0))

        outv = ninf
        outi = jnp.zeros((16,), jnp.int32)
        for t in range(K):
            gs = [gmax_v[pl.ds(v * 16, 16)] for v in range(NGPAD // 16)]
            gacc = gs[0]
            for gv in gs[1:]:
                gacc = jnp.maximum(gacc, gv)
            gm = jnp.max(gacc)
            cacc = jnp.full((16,), _BIG_I32, jnp.int32)
            for v, gv in enumerate(gs):
                cacc = jnp.minimum(
                    cacc, jnp.where(gv == gm, iota + v * 16, _BIG_I32))
            gstar = jnp.min(cacc)

            def rescan(cc, carry):
                m, mc = carry
                v = buf_v[pl.ds(gstar * GRP * 16 + cc * 16, 16)]
                upd = v > m
                return (jnp.where(upd, v, m), jnp.where(upd, cc, mc))

            m, mc = lax.fori_loop(0, GRP, rescan,
                                  (ninf, jnp.zeros((16,), jnp.int32)))
            gm2 = jnp.max(m)
            pe = jnp.min(jnp.where(
                m == gm2, (gstar * GRP + mc) * 16 + iota, _BIG_I32))
            outv = jnp.where(iota == t, gm2, outv)
            outi = jnp.where(iota == t, base + pe, outi)

            pb = (pe // 16) * 16
            cur = buf_v[pl.ds(pb, 16)]
            buf_v[pl.ds(pb, 16)] = jnp.where(iota == pe - pb, _NEG_INF, cur)

            ngm = group_max(gstar)
            gb = (gstar // 16) * 16
            cur = gmax_v[pl.ds(gb, 16)]
            gmax_v[pl.ds(gb, 16)] = jnp.where(iota == gstar - gb, ngm, cur)

        ov_v[...] = outv
        oi_v[...] = outi
        pltpu.sync_copy(ov_v, cv_hbm.at[pl.ds(wid * K, K)])
        pltpu.sync_copy(oi_v, ci_hbm.at[pl.ds(wid * K, K)])

    return topk_local(sims)


def _sc_topk_merge(cand_v, cand_i):
    mesh = plsc.VectorSubcoreMesh(core_axis_name="c", subcore_axis_name="s")
    NC = NW * K // 16

    @functools.partial(
        pl.kernel,
        mesh=mesh,
        out_type=[
            jax.ShapeDtypeStruct((K,), jnp.float32),
            jax.ShapeDtypeStruct((K,), jnp.int32),
        ],
        scratch_types=[
            pltpu.VMEM((NW * K,), jnp.float32),
            pltpu.VMEM((NW * K,), jnp.int32),
            pltpu.VMEM((K,), jnp.float32),
            pltpu.VMEM((K,), jnp.int32),
        ],
        compiler_params=pltpu.CompilerParams(needs_layout_passes=False),
    )
    def topk_merge(cv_hbm, ci_hbm, tv_hbm, ti_hbm, cv_v, ci_v, ov_v, oi_v):
        c = lax.axis_index("c")
        s = lax.axis_index("s")
        wid = s * 2 + c

        @pl.when(wid == 0)
        def _():
            pltpu.sync_copy(cv_hbm, cv_v)
            pltpu.sync_copy(ci_hbm, ci_v)
            iota = lax.iota(jnp.int32, 16)
            ninf = jnp.full((16,), _NEG_INF, jnp.float32)
            outv = ninf
            outi = jnp.zeros((16,), jnp.int32)
            for t in range(K):
                def vmax(cc, macc):
                    return jnp.maximum(macc, cv_v[pl.ds(cc * 16, 16)])
                gm = jnp.max(lax.fori_loop(0, NC, vmax, ninf))

                def imin(cc, micc):
                    vv = cv_v[pl.ds(cc * 16, 16)]
                    iv = ci_v[pl.ds(cc * 16, 16)]
                    return jnp.minimum(micc, jnp.where(vv == gm, iv, _BIG_I32))
                gi = jnp.min(lax.fori_loop(
                    0, NC, imin, jnp.full((16,), _BIG_I32, jnp.int32)))

                def mask(cc, carry):
                    vv = cv_v[pl.ds(cc * 16, 16)]
                    iv = ci_v[pl.ds(cc * 16, 16)]
                    hit = jnp.logical_and(vv == gm, iv == gi)
                    cv_v[pl.ds(cc * 16, 16)] = jnp.where(hit, _NEG_INF, vv)
                    return carry
                lax.fori_loop(0, NC, mask, jnp.int32(0))

                outv = jnp.where(iota == t, gm, outv)
                outi = jnp.where(iota == t, gi, outi)
            ov_v[...] = outv
            oi_v[...] = outi
            pltpu.sync_copy(ov_v, tv_hbm)
            pltpu.sync_copy(oi_v, ti_hbm)

    return topk_merge(cand_v, cand_i)


def _gather_body(idx_sref, vblk_ref, out_ref):
    j = pl.program_id(0)
    r = idx_sref[j]
    lane = lax.broadcasted_iota(jnp.int32, (D_MODEL, 128), 1)
    col = jnp.sum(jnp.where(lane == r % 128, vblk_ref[...], 0.0),
                  axis=1, keepdims=True)
    outlane = lax.broadcasted_iota(jnp.int32, (D_MODEL, K), 1)
    out_ref[...] = jnp.where(outlane == j, col, out_ref[...])


def _gather_cols(values_t, idx):
    grid_spec = pltpu.PrefetchScalarGridSpec(
        num_scalar_prefetch=1,
        grid=(K,),
        in_specs=[
            pl.BlockSpec((D_MODEL, 128), lambda j, idx_ref: (0, idx_ref[j] // 128)),
        ],
        out_specs=pl.BlockSpec((D_MODEL, K), lambda j, idx_ref: (0, 0)),
    )
    return pl.pallas_call(
        _gather_body,
        grid_spec=grid_spec,
        out_shape=jax.ShapeDtypeStruct((D_MODEL, K), jnp.float32),
    )(idx, values_t)


def kernel(query, keys, values, k):
    sims = _sims(query, keys.T)
    cand_v, cand_i = _sc_topk_local(sims)
    topk_sims, topk_idx = _sc_topk_merge(cand_v, cand_i)
    valid = jnp.arange(K) < k
    topk_sims = jnp.where(valid, topk_sims, topk_sims[0])
    topk_idx = jnp.where(valid, topk_idx, topk_idx[0])
    vals_t = _gather_cols(values.T, topk_idx)
    return vals_t.T, topk_sims


# speedup vs baseline: 30.5319x; 1.1032x over previous
import functools

import jax
import jax.numpy as jnp
from jax import lax
from jax.experimental import pallas as pl
from jax.experimental.pallas import tpu as pltpu
from jax.experimental.pallas import tpu_sc as plsc

D_MODEL = 64
N = 1000000
K = 16
BLK = 32768
G = -(-N // BLK)
TOT = G * BLK
NW = 32
CH = TOT // NW
GRP = 62
NGRP = CH // (GRP * 16)
NGPAD = NGRP

_NEG_INF = float("-inf")
_BIG_I32 = 2**31 - 1


def _sims_body(q_ref, k_ref, s_out):
    i = pl.program_id(0)
    q = q_ref[...]
    qn = q / jnp.maximum(jnp.sqrt(jnp.sum(q * q)), 1e-12)
    kblk = k_ref[...]
    dots = jnp.sum(kblk * qn, axis=0, keepdims=True)
    nrm2 = jnp.sum(kblk * kblk, axis=0, keepdims=True)
    s3 = dots / jnp.maximum(jnp.sqrt(nrm2), 1e-12)
    lin = lax.broadcasted_iota(jnp.int32, (1, BLK), 1)
    s3 = jnp.where(i * BLK + lin < N, s3, _NEG_INF)
    s_out[...] = s3.reshape(BLK)


def _sims(query, keys_t):
    return pl.pallas_call(
        _sims_body,
        grid=(G,),
        in_specs=[
            pl.BlockSpec((D_MODEL, 1), lambda i: (0, 0)),
            pl.BlockSpec((D_MODEL, BLK), lambda i: (0, i)),
        ],
        out_specs=pl.BlockSpec((BLK,), lambda i: (i,)),
        out_shape=jax.ShapeDtypeStruct((TOT,), jnp.float32),
    )(query.reshape(D_MODEL, 1), keys_t)


def _sc_topk_local(sims):
    mesh = plsc.VectorSubcoreMesh(core_axis_name="c", subcore_axis_name="s")

    @functools.partial(
        pl.kernel,
        mesh=mesh,
        out_type=[
            jax.ShapeDtypeStruct((NW * K,), jnp.float32),
            jax.ShapeDtypeStruct((NW * K,), jnp.int32),
        ],
        scratch_types=[
            pltpu.VMEM((CH,), jnp.float32),
            pltpu.VMEM((NGPAD,), jnp.float32),
            pltpu.VMEM((K,), jnp.float32),
            pltpu.VMEM((K,), jnp.int32),
        ],
        compiler_params=pltpu.CompilerParams(needs_layout_passes=False),
    )
    def topk_local(sims_hbm, cv_hbm, ci_hbm, buf_v, gmax_v, ov_v, oi_v):
        c = lax.axis_index("c")
        s = lax.axis_index("s")
        wid = s * 2 + c
        base = wid * CH
        pltpu.sync_copy(sims_hbm.at[pl.ds(base, CH)], buf_v)

        iota = lax.iota(jnp.int32, 16)
        ninf = jnp.full((16,), _NEG_INF, jnp.float32)

        for v in range(NGPAD // 16):
            gmax_v[pl.ds(v * 16, 16)] = ninf

        def group_max(g):
            def inner(cc, macc):
                return jnp.maximum(macc, buf_v[pl.ds(g * GRP * 16 + cc * 16, 16)])
            return jnp.max(lax.fori_loop(0, GRP, inner, ninf))

        def phase1(g, carry):
            gm = group_max(g)
            gb = (g // 16) * 16
            cur = gmax_v[pl.ds(gb, 16)]
            gmax_v[pl.ds(gb, 16)] = jnp.where(iota == g - gb, gm, cur)
            return carry

        lax.fori_loop(0, NGRP, phase1, jnp.int32(0))

        outv = ninf
        outi = jnp.zeros((16,), jnp.int32)
        for t in range(K):
            gs = [gmax_v[pl.ds(v * 16, 16)] for v in range(NGPAD // 16)]
            gacc = gs[0]
            for gv in gs[1:]:
                gacc = jnp.maximum(gacc, gv)
            gm = jnp.max(gacc)
            cacc = jnp.full((16,), _BIG_I32, jnp.int32)
            for v, gv in enumerate(gs):
                cacc = jnp.minimum(
                    cacc, jnp.where(gv == gm, iota + v * 16, _BIG_I32))
            gstar = jnp.min(cacc)

            def rescan(cc, carry):
                m, mc = carry
                v = buf_v[pl.ds(gstar * GRP * 16 + cc * 16, 16)]
                upd = v > m
                return (jnp.where(upd, v, m), jnp.where(upd, cc, mc))

            m, mc = lax.fori_loop(0, GRP, rescan,
                                  (ninf, jnp.zeros((16,), jnp.int32)))
            gm2 = jnp.max(m)
            pe = jnp.min(jnp.where(
                m == gm2, (gstar * GRP + mc) * 16 + iota, _BIG_I32))
            outv = jnp.where(iota == t, gm2, outv)
            outi = jnp.where(iota == t, base + pe, outi)

            pb = (pe // 16) * 16
            cur = buf_v[pl.ds(pb, 16)]
            buf_v[pl.ds(pb, 16)] = jnp.where(iota == pe - pb, _NEG_INF, cur)

            ngm = group_max(gstar)
            gb = (gstar // 16) * 16
            cur = gmax_v[pl.ds(gb, 16)]
            gmax_v[pl.ds(gb, 16)] = jnp.where(iota == gstar - gb, ngm, cur)

        ov_v[...] = outv
        oi_v[...] = outi
        pltpu.sync_copy(ov_v, cv_hbm.at[pl.ds(wid * K, K)])
        pltpu.sync_copy(oi_v, ci_hbm.at[pl.ds(wid * K, K)])

    return topk_local(sims)


def _sc_topk_merge(cand_v, cand_i):
    mesh = plsc.VectorSubcoreMesh(core_axis_name="c", subcore_axis_name="s")
    NC = NW * K // 16

    @functools.partial(
        pl.kernel,
        mesh=mesh,
        out_type=[
            jax.ShapeDtypeStruct((K,), jnp.float32),
            jax.ShapeDtypeStruct((K,), jnp.int32),
        ],
        scratch_types=[
            pltpu.VMEM((NW * K,), jnp.float32),
            pltpu.VMEM((NW * K,), jnp.int32),
            pltpu.VMEM((K,), jnp.float32),
            pltpu.VMEM((K,), jnp.int32),
        ],
        compiler_params=pltpu.CompilerParams(needs_layout_passes=False),
    )
    def topk_merge(cv_hbm, ci_hbm, tv_hbm, ti_hbm, cv_v, ci_v, ov_v, oi_v):
        c = lax.axis_index("c")
        s = lax.axis_index("s")
        wid = s * 2 + c

        @pl.when(wid == 0)
        def _():
            pltpu.sync_copy(cv_hbm, cv_v)
            pltpu.sync_copy(ci_hbm, ci_v)
            iota = lax.iota(jnp.int32, 16)
            ninf = jnp.full((16,), _NEG_INF, jnp.float32)
            outv = ninf
            outi = jnp.zeros((16,), jnp.int32)
            for t in range(K):
                def vmax(cc, macc):
                    return jnp.maximum(macc, cv_v[pl.ds(cc * 16, 16)])
                gm = jnp.max(lax.fori_loop(0, NC, vmax, ninf))

                def imin(cc, micc):
                    vv = cv_v[pl.ds(cc * 16, 16)]
                    iv = ci_v[pl.ds(cc * 16, 16)]
                    return jnp.minimum(micc, jnp.where(vv == gm, iv, _BIG_I32))
                gi = jnp.min(lax.fori_loop(
                    0, NC, imin, jnp.full((16,), _BIG_I32, jnp.int32)))

                def mask(cc, carry):
                    vv = cv_v[pl.ds(cc * 16, 16)]
                    iv = ci_v[pl.ds(cc * 16, 16)]
                    hit = jnp.logical_and(vv == gm, iv == gi)
                    cv_v[pl.ds(cc * 16, 16)] = jnp.where(hit, _NEG_INF, vv)
                    return carry
                lax.fori_loop(0, NC, mask, jnp.int32(0))

                outv = jnp.where(iota == t, gm, outv)
                outi = jnp.where(iota == t, gi, outi)
            ov_v[...] = outv
            oi_v[...] = outi
            pltpu.sync_copy(ov_v, tv_hbm)
            pltpu.sync_copy(oi_v, ti_hbm)

    return topk_merge(cand_v, cand_i)


def _gather_body(idx_sref, vblk_ref, out_ref):
    j = pl.program_id(0)
    r = idx_sref[j]
    lane = lax.broadcasted_iota(jnp.int32, (D_MODEL, 128), 1)
    col = jnp.sum(jnp.where(lane == r % 128, vblk_ref[...], 0.0),
                  axis=1, keepdims=True)
    outlane = lax.broadcasted_iota(jnp.int32, (D_MODEL, K), 1)
    out_ref[...] = jnp.where(outlane == j, col, out_ref[...])


def _gather_cols(values_t, idx):
    grid_spec = pltpu.PrefetchScalarGridSpec(
        num_scalar_prefetch=1,
        grid=(K,),
        in_specs=[
            pl.BlockSpec((D_MODEL, 128), lambda j, idx_ref: (0, idx_ref[j] // 128)),
        ],
        out_specs=pl.BlockSpec((D_MODEL, K), lambda j, idx_ref: (0, 0)),
    )
    return pl.pallas_call(
        _gather_body,
        grid_spec=grid_spec,
        out_shape=jax.ShapeDtypeStruct((D_MODEL, K), jnp.float32),
    )(idx, values_t)


def kernel(query, keys, values, k):
    sims = _sims(query, keys.T)
    cand_v, cand_i = _sc_topk_local(sims)
    topk_sims, topk_idx = _sc_topk_merge(cand_v, cand_i)
    valid = jnp.arange(K) < k
    topk_sims = jnp.where(valid, topk_sims, topk_sims[0])
    topk_idx = jnp.where(valid, topk_idx, topk_idx[0])
    vals_t = _gather_cols(values.T, topk_idx)
    return vals_t.T, topk_sims


# speedup vs baseline: 31.0797x; 1.0179x over previous
import functools

import jax
import jax.numpy as jnp
from jax import lax
from jax.experimental import pallas as pl
from jax.experimental.pallas import tpu as pltpu
from jax.experimental.pallas import tpu_sc as plsc

D_MODEL = 64
N = 1000000
K = 16
BLK = 65536
G = -(-N // BLK)
TOT = G * BLK
NW = 32
CH = TOT // NW
GRP = 64
NGRP = CH // (GRP * 16)
NGPAD = NGRP

_NEG_INF = float("-inf")
_BIG_I32 = 2**31 - 1


def _sims_body(q_ref, k_ref, s_out):
    i = pl.program_id(0)
    q = q_ref[...]
    qn = q / jnp.maximum(jnp.sqrt(jnp.sum(q * q)), 1e-12)
    kblk = k_ref[...]
    dots = jnp.sum(kblk * qn, axis=0, keepdims=True)
    nrm2 = jnp.sum(kblk * kblk, axis=0, keepdims=True)
    s3 = dots / jnp.maximum(jnp.sqrt(nrm2), 1e-12)
    lin = lax.broadcasted_iota(jnp.int32, (1, BLK), 1)
    s3 = jnp.where(i * BLK + lin < N, s3, _NEG_INF)
    s_out[...] = s3.reshape(BLK)


def _sims(query, keys_t):
    return pl.pallas_call(
        _sims_body,
        grid=(G,),
        in_specs=[
            pl.BlockSpec((D_MODEL, 1), lambda i: (0, 0)),
            pl.BlockSpec((D_MODEL, BLK), lambda i: (0, i)),
        ],
        out_specs=pl.BlockSpec((BLK,), lambda i: (i,)),
        out_shape=jax.ShapeDtypeStruct((TOT,), jnp.float32),
    )(query.reshape(D_MODEL, 1), keys_t)


def _sc_topk_local(sims):
    mesh = plsc.VectorSubcoreMesh(core_axis_name="c", subcore_axis_name="s")

    @functools.partial(
        pl.kernel,
        mesh=mesh,
        out_type=[
            jax.ShapeDtypeStruct((NW * K,), jnp.float32),
            jax.ShapeDtypeStruct((NW * K,), jnp.int32),
        ],
        scratch_types=[
            pltpu.VMEM((CH,), jnp.float32),
            pltpu.VMEM((NGPAD,), jnp.float32),
            pltpu.VMEM((K,), jnp.float32),
            pltpu.VMEM((K,), jnp.int32),
        ],
        compiler_params=pltpu.CompilerParams(needs_layout_passes=False),
    )
    def topk_local(sims_hbm, cv_hbm, ci_hbm, buf_v, gmax_v, ov_v, oi_v):
        c = lax.axis_index("c")
        s = lax.axis_index("s")
        wid = s * 2 + c
        base = wid * CH
        pltpu.sync_copy(sims_hbm.at[pl.ds(base, CH)], buf_v)

        iota = lax.iota(jnp.int32, 16)
        ninf = jnp.full((16,), _NEG_INF, jnp.float32)

        for v in range(NGPAD // 16):
            gmax_v[pl.ds(v * 16, 16)] = ninf

        def group_max(g):
            def inner(cc, macc):
                return jnp.maximum(macc, buf_v[pl.ds(g * GRP * 16 + cc * 16, 16)])
            return jnp.max(lax.fori_loop(0, GRP, inner, ninf))

        def phase1(g, carry):
            gm = group_max(g)
            gb = (g // 16) * 16
            cur = gmax_v[pl.ds(gb, 16)]
            gmax_v[pl.ds(gb, 16)] = jnp.where(iota == g - gb, gm, cur)
            return carry

        lax.fori_loop(0, NGRP, phase1, jnp.int32(0))

        outv = ninf
        outi = jnp.zeros((16,), jnp.int32)
        for t in range(K):
            gs = [gmax_v[pl.ds(v * 16, 16)] for v in range(NGPAD // 16)]
            gacc = gs[0]
            for gv in gs[1:]:
                gacc = jnp.maximum(gacc, gv)
            gm = jnp.max(gacc)
            cacc = jnp.full((16,), _BIG_I32, jnp.int32)
            for v, gv in enumerate(gs):
                cacc = jnp.minimum(
                    cacc, jnp.where(gv == gm, iota + v * 16, _BIG_I32))
            gstar = jnp.min(cacc)

            def rescan(cc, carry):
                m, mc = carry
                v = buf_v[pl.ds(gstar * GRP * 16 + cc * 16, 16)]
                upd = v > m
                return (jnp.where(upd, v, m), jnp.where(upd, cc, mc))

            m, mc = lax.fori_loop(0, GRP, rescan,
                                  (ninf, jnp.zeros((16,), jnp.int32)))
            gm2 = jnp.max(m)
            pe = jnp.min(jnp.where(
                m == gm2, (gstar * GRP + mc) * 16 + iota, _BIG_I32))
            outv = jnp.where(iota == t, gm2, outv)
            outi = jnp.where(iota == t, base + pe, outi)

            pb = (pe // 16) * 16
            cur = buf_v[pl.ds(pb, 16)]
            buf_v[pl.ds(pb, 16)] = jnp.where(iota == pe - pb, _NEG_INF, cur)

            ngm = group_max(gstar)
            gb = (gstar // 16) * 16
            cur = gmax_v[pl.ds(gb, 16)]
            gmax_v[pl.ds(gb, 16)] = jnp.where(iota == gstar - gb, ngm, cur)

        ov_v[...] = outv
        oi_v[...] = outi
        pltpu.sync_copy(ov_v, cv_hbm.at[pl.ds(wid * K, K)])
        pltpu.sync_copy(oi_v, ci_hbm.at[pl.ds(wid * K, K)])

    return topk_local(sims)


def _sc_topk_merge(cand_v, cand_i):
    mesh = plsc.VectorSubcoreMesh(core_axis_name="c", subcore_axis_name="s")
    NC = NW * K // 16

    @functools.partial(
        pl.kernel,
        mesh=mesh,
        out_type=[
            jax.ShapeDtypeStruct((K,), jnp.float32),
            jax.ShapeDtypeStruct((K,), jnp.int32),
        ],
        scratch_types=[
            pltpu.VMEM((NW * K,), jnp.float32),
            pltpu.VMEM((NW * K,), jnp.int32),
            pltpu.VMEM((K,), jnp.float32),
            pltpu.VMEM((K,), jnp.int32),
        ],
        compiler_params=pltpu.CompilerParams(needs_layout_passes=False),
    )
    def topk_merge(cv_hbm, ci_hbm, tv_hbm, ti_hbm, cv_v, ci_v, ov_v, oi_v):
        c = lax.axis_index("c")
        s = lax.axis_index("s")
        wid = s * 2 + c

        @pl.when(wid == 0)
        def _():
            pltpu.sync_copy(cv_hbm, cv_v)
            pltpu.sync_copy(ci_hbm, ci_v)
            iota = lax.iota(jnp.int32, 16)
            ninf = jnp.full((16,), _NEG_INF, jnp.float32)
            outv = ninf
            outi = jnp.zeros((16,), jnp.int32)
            for t in range(K):
                def vmax(cc, macc):
                    return jnp.maximum(macc, cv_v[pl.ds(cc * 16, 16)])
                gm = jnp.max(lax.fori_loop(0, NC, vmax, ninf))

                def imin(cc, micc):
                    vv = cv_v[pl.ds(cc * 16, 16)]
                    iv = ci_v[pl.ds(cc * 16, 16)]
                    return jnp.minimum(micc, jnp.where(vv == gm, iv, _BIG_I32))
                gi = jnp.min(lax.fori_loop(
                    0, NC, imin, jnp.full((16,), _BIG_I32, jnp.int32)))

                def mask(cc, carry):
                    vv = cv_v[pl.ds(cc * 16, 16)]
                    iv = ci_v[pl.ds(cc * 16, 16)]
                    hit = jnp.logical_and(vv == gm, iv == gi)
                    cv_v[pl.ds(cc * 16, 16)] = jnp.where(hit, _NEG_INF, vv)
                    return carry
                lax.fori_loop(0, NC, mask, jnp.int32(0))

                outv = jnp.where(iota == t, gm, outv)
                outi = jnp.where(iota == t, gi, outi)
            ov_v[...] = outv
            oi_v[...] = outi
            pltpu.sync_copy(ov_v, tv_hbm)
            pltpu.sync_copy(oi_v, ti_hbm)

    return topk_merge(cand_v, cand_i)


def _gather_body(idx_sref, vblk_ref, out_ref):
    j = pl.program_id(0)
    r = idx_sref[j]
    lane = lax.broadcasted_iota(jnp.int32, (D_MODEL, 128), 1)
    col = jnp.sum(jnp.where(lane == r % 128, vblk_ref[...], 0.0),
                  axis=1, keepdims=True)
    outlane = lax.broadcasted_iota(jnp.int32, (D_MODEL, K), 1)
    out_ref[...] = jnp.where(outlane == j, col, out_ref[...])


def _gather_cols(values_t, idx):
    grid_spec = pltpu.PrefetchScalarGridSpec(
        num_scalar_prefetch=1,
        grid=(K,),
        in_specs=[
            pl.BlockSpec((D_MODEL, 128), lambda j, idx_ref: (0, idx_ref[j] // 128)),
        ],
        out_specs=pl.BlockSpec((D_MODEL, K), lambda j, idx_ref: (0, 0)),
    )
    return pl.pallas_call(
        _gather_body,
        grid_spec=grid_spec,
        out_shape=jax.ShapeDtypeStruct((D_MODEL, K), jnp.float32),
    )(idx, values_t)


def kernel(query, keys, values, k):
    sims = _sims(query, keys.T)
    cand_v, cand_i = _sc_topk_local(sims)
    topk_sims, topk_idx = _sc_topk_merge(cand_v, cand_i)
    valid = jnp.arange(K) < k
    topk_sims = jnp.where(valid, topk_sims, topk_sims[0])
    topk_idx = jnp.where(valid, topk_idx, topk_idx[0])
    vals_t = _gather_cols(values.T, topk_idx)
    return vals_t.T, topk_sims


# speedup vs baseline: 32.9315x; 1.0596x over previous
import functools

import jax
import jax.numpy as jnp
from jax import lax
from jax.experimental import pallas as pl
from jax.experimental.pallas import tpu as pltpu
from jax.experimental.pallas import tpu_sc as plsc

D_MODEL = 64
N = 1000000
K = 16
BLK = 65536
G = -(-N // BLK)
TOT = G * BLK
NW = 32
CH = TOT // NW
GRP = 64
NGRP = CH // (GRP * 16)
NGPAD = NGRP

_NEG_INF = float("-inf")
_BIG_I32 = 2**31 - 1


def _sims_body(q_ref, k_ref, s_out):
    i = pl.program_id(0)
    q = q_ref[...]
    qn = q / jnp.maximum(jnp.sqrt(jnp.sum(q * q)), 1e-12)
    kblk = k_ref[...]
    dots = jnp.sum(kblk * qn, axis=0, keepdims=True)
    nrm2 = jnp.sum(kblk * kblk, axis=0, keepdims=True)
    s3 = dots / jnp.maximum(jnp.sqrt(nrm2), 1e-12)
    lin = lax.broadcasted_iota(jnp.int32, (1, BLK), 1)
    s3 = jnp.where(i * BLK + lin < N, s3, _NEG_INF)
    s_out[...] = s3.reshape(BLK)


def _sims(query, keys_t):
    return pl.pallas_call(
        _sims_body,
        grid=(G,),
        in_specs=[
            pl.BlockSpec((D_MODEL, 1), lambda i: (0, 0)),
            pl.BlockSpec((D_MODEL, BLK), lambda i: (0, i)),
        ],
        out_specs=pl.BlockSpec((BLK,), lambda i: (i,)),
        out_shape=jax.ShapeDtypeStruct((TOT,), jnp.float32),
    )(query.reshape(D_MODEL, 1), keys_t)


def _sc_topk_local(sims):
    mesh = plsc.VectorSubcoreMesh(core_axis_name="c", subcore_axis_name="s")

    @functools.partial(
        pl.kernel,
        mesh=mesh,
        out_type=[
            jax.ShapeDtypeStruct((NW * K,), jnp.float32),
            jax.ShapeDtypeStruct((NW * K,), jnp.int32),
        ],
        scratch_types=[
            pltpu.VMEM((CH,), jnp.float32),
            pltpu.VMEM((NGPAD,), jnp.float32),
            pltpu.VMEM((K,), jnp.float32),
            pltpu.VMEM((K,), jnp.int32),
        ],
        compiler_params=pltpu.CompilerParams(needs_layout_passes=False),
    )
    def topk_local(sims_hbm, cv_hbm, ci_hbm, buf_v, gmax_v, ov_v, oi_v):
        c = lax.axis_index("c")
        s = lax.axis_index("s")
        wid = s * 2 + c
        base = wid * CH
        pltpu.sync_copy(sims_hbm.at[pl.ds(base, CH)], buf_v)

        iota = lax.iota(jnp.int32, 16)
        ninf = jnp.full((16,), _NEG_INF, jnp.float32)

        for v in range(NGPAD // 16):
            gmax_v[pl.ds(v * 16, 16)] = ninf

        def group_max(g):
            def inner(cc, macc):
                base = g * GRP * 16 + cc * 64
                for u in range(4):
                    macc = jnp.maximum(macc, buf_v[pl.ds(base + u * 16, 16)])
                return macc
            return jnp.max(lax.fori_loop(0, GRP // 4, inner, ninf))

        def phase1(g, carry):
            gm = group_max(g)
            gb = (g // 16) * 16
            cur = gmax_v[pl.ds(gb, 16)]
            gmax_v[pl.ds(gb, 16)] = jnp.where(iota == g - gb, gm, cur)
            return carry

        lax.fori_loop(0, NGRP, phase1, jnp.int32(0))

        outv = ninf
        outi = jnp.zeros((16,), jnp.int32)
        for t in range(K):
            gs = [gmax_v[pl.ds(v * 16, 16)] for v in range(NGPAD // 16)]
            gacc = gs[0]
            for gv in gs[1:]:
                gacc = jnp.maximum(gacc, gv)
            gm = jnp.max(gacc)
            cacc = jnp.full((16,), _BIG_I32, jnp.int32)
            for v, gv in enumerate(gs):
                cacc = jnp.minimum(
                    cacc, jnp.where(gv == gm, iota + v * 16, _BIG_I32))
            gstar = jnp.min(cacc)

            def rescan(cc, carry):
                m, mc = carry
                for u in range(4):
                    v = buf_v[pl.ds(gstar * GRP * 16 + (cc * 4 + u) * 16, 16)]
                    upd = v > m
                    m = jnp.where(upd, v, m)
                    mc = jnp.where(upd, cc * 4 + u, mc)
                return (m, mc)

            m, mc = lax.fori_loop(0, GRP // 4, rescan,
                                  (ninf, jnp.zeros((16,), jnp.int32)))
            gm2 = jnp.max(m)
            pe = jnp.min(jnp.where(
                m == gm2, (gstar * GRP + mc) * 16 + iota, _BIG_I32))
            outv = jnp.where(iota == t, gm2, outv)
            outi = jnp.where(iota == t, base + pe, outi)

            pb = (pe // 16) * 16
            cur = buf_v[pl.ds(pb, 16)]
            buf_v[pl.ds(pb, 16)] = jnp.where(iota == pe - pb, _NEG_INF, cur)

            ngm = group_max(gstar)
            gb = (gstar // 16) * 16
            cur = gmax_v[pl.ds(gb, 16)]
            gmax_v[pl.ds(gb, 16)] = jnp.where(iota == gstar - gb, ngm, cur)

        ov_v[...] = outv
        oi_v[...] = outi
        pltpu.sync_copy(ov_v, cv_hbm.at[pl.ds(wid * K, K)])
        pltpu.sync_copy(oi_v, ci_hbm.at[pl.ds(wid * K, K)])

    return topk_local(sims)


def _sc_topk_merge(cand_v, cand_i):
    mesh = plsc.VectorSubcoreMesh(core_axis_name="c", subcore_axis_name="s")
    NC = NW * K // 16

    @functools.partial(
        pl.kernel,
        mesh=mesh,
        out_type=[
            jax.ShapeDtypeStruct((K,), jnp.float32),
            jax.ShapeDtypeStruct((K,), jnp.int32),
        ],
        scratch_types=[
            pltpu.VMEM((NW * K,), jnp.float32),
            pltpu.VMEM((NW * K,), jnp.int32),
            pltpu.VMEM((K,), jnp.float32),
            pltpu.VMEM((K,), jnp.int32),
        ],
        compiler_params=pltpu.CompilerParams(needs_layout_passes=False),
    )
    def topk_merge(cv_hbm, ci_hbm, tv_hbm, ti_hbm, cv_v, ci_v, ov_v, oi_v):
        c = lax.axis_index("c")
        s = lax.axis_index("s")
        wid = s * 2 + c

        @pl.when(wid == 0)
        def _():
            pltpu.sync_copy(cv_hbm, cv_v)
            pltpu.sync_copy(ci_hbm, ci_v)
            iota = lax.iota(jnp.int32, 16)
            ninf = jnp.full((16,), _NEG_INF, jnp.float32)
            outv = ninf
            outi = jnp.zeros((16,), jnp.int32)
            for t in range(K):
                def vmax(cc, macc):
                    for u in range(4):
                        macc = jnp.maximum(
                            macc, cv_v[pl.ds((cc * 4 + u) * 16, 16)])
                    return macc
                gm = jnp.max(lax.fori_loop(0, NC // 4, vmax, ninf))

                def imin(cc, micc):
                    for u in range(4):
                        vv = cv_v[pl.ds((cc * 4 + u) * 16, 16)]
                        iv = ci_v[pl.ds((cc * 4 + u) * 16, 16)]
                        micc = jnp.minimum(
                            micc, jnp.where(vv == gm, iv, _BIG_I32))
                    return micc
                gi = jnp.min(lax.fori_loop(
                    0, NC // 4, imin, jnp.full((16,), _BIG_I32, jnp.int32)))

                def mask(cc, carry):
                    for u in range(4):
                        vv = cv_v[pl.ds((cc * 4 + u) * 16, 16)]
                        iv = ci_v[pl.ds((cc * 4 + u) * 16, 16)]
                        hit = jnp.logical_and(vv == gm, iv == gi)
                        cv_v[pl.ds((cc * 4 + u) * 16, 16)] = jnp.where(
                            hit, _NEG_INF, vv)
                    return carry
                lax.fori_loop(0, NC // 4, mask, jnp.int32(0))

                outv = jnp.where(iota == t, gm, outv)
                outi = jnp.where(iota == t, gi, outi)
            ov_v[...] = outv
            oi_v[...] = outi
            pltpu.sync_copy(ov_v, tv_hbm)
            pltpu.sync_copy(oi_v, ti_hbm)

    return topk_merge(cand_v, cand_i)


def _gather_body(idx_sref, vblk_ref, out_ref):
    j = pl.program_id(0)
    r = idx_sref[j]
    lane = lax.broadcasted_iota(jnp.int32, (D_MODEL, 128), 1)
    col = jnp.sum(jnp.where(lane == r % 128, vblk_ref[...], 0.0),
                  axis=1, keepdims=True)
    outlane = lax.broadcasted_iota(jnp.int32, (D_MODEL, K), 1)
    out_ref[...] = jnp.where(outlane == j, col, out_ref[...])


def _gather_cols(values_t, idx):
    grid_spec = pltpu.PrefetchScalarGridSpec(
        num_scalar_prefetch=1,
        grid=(K,),
        in_specs=[
            pl.BlockSpec((D_MODEL, 128), lambda j, idx_ref: (0, idx_ref[j] // 128)),
        ],
        out_specs=pl.BlockSpec((D_MODEL, K), lambda j, idx_ref: (0, 0)),
    )
    return pl.pallas_call(
        _gather_body,
        grid_spec=grid_spec,
        out_shape=jax.ShapeDtypeStruct((D_MODEL, K), jnp.float32),
    )(idx, values_t)


def kernel(query, keys, values, k):
    sims = _sims(query, keys.T)
    cand_v, cand_i = _sc_topk_local(sims)
    topk_sims, topk_idx = _sc_topk_merge(cand_v, cand_i)
    valid = jnp.arange(K) < k
    topk_sims = jnp.where(valid, topk_sims, topk_sims[0])
    topk_idx = jnp.where(valid, topk_idx, topk_idx[0])
    vals_t = _gather_cols(values.T, topk_idx)
    return vals_t.T, topk_sims
